# baseline (speedup 1.0000x reference)
import functools

import jax
import jax.numpy as jnp
from jax import lax
from jax.experimental import pallas as pl
from jax.experimental.pallas import tpu as pltpu
from jax.experimental.pallas import tpu_sc as plsc

_SC_INFO = plsc.get_sparse_core_info()
_NC = _SC_INFO.num_cores
_NS = _SC_INFO.num_subcores
_NW = _NC * _NS


def _idx_body(n, ncols, noise_ref, out_ref):
    x = noise_ref[...]
    unif = 0.5 * (1.0 + lax.erf(x / jnp.sqrt(jnp.asarray(2.0, x.dtype))))
    idx = jnp.floor(unif * n).astype(jnp.int32)
    idx = jnp.minimum(idx, n - 1)
    col = lax.broadcasted_iota(jnp.int32, x.shape, 1)
    out_ref[...] = idx * ncols + col


def _flat_indices(noise, n, ncols):
    b = noise.shape[0]
    block_rows = 16384
    grid = (b // block_rows,)
    return pl.pallas_call(
        functools.partial(_idx_body, n, ncols),
        grid=grid,
        in_specs=[pl.BlockSpec((block_rows, ncols), lambda i: (i, 0))],
        out_specs=pl.BlockSpec((block_rows, ncols), lambda i: (i, 0)),
        out_shape=jax.ShapeDtypeStruct((b, ncols), jnp.int32),
    )(noise)


def _make_sc_gather(total, chunk):
    per_w = total // _NW
    n_chunks = per_w // chunk
    mesh = plsc.VectorSubcoreMesh(core_axis_name="c", subcore_axis_name="s")

    nbuf = 4
    depth = 2

    @functools.partial(
        pl.kernel,
        mesh=mesh,
        out_type=jax.ShapeDtypeStruct((total,), jnp.float32),
        scratch_types=(
            [pltpu.VMEM((chunk,), jnp.int32) for _ in range(nbuf)]
            + [pltpu.VMEM((chunk,), jnp.float32) for _ in range(nbuf)]
            + [pltpu.SemaphoreType.DMA for _ in range(2 * nbuf)]
        ),
    )
    def sc_gather(order_hbm, idx_hbm, out_hbm, *bufs):
        idx_bufs = bufs[:nbuf]
        row_bufs = bufs[nbuf:2 * nbuf]
        gsems = bufs[2 * nbuf:3 * nbuf]
        osems = bufs[3 * nbuf:]
        wid = lax.axis_index("s") * _NC + lax.axis_index("c")
        base = wid * per_w
        gathers = [None] * nbuf
        stores = [None] * nbuf
        for k in range(n_chunks):
            s = k % nbuf
            if stores[s] is not None:
                stores[s].wait()
            pltpu.sync_copy(idx_hbm.at[pl.ds(base + k * chunk, chunk)],
                            idx_bufs[s])
            gathers[s] = pltpu.async_copy(
                order_hbm.at[idx_bufs[s]], row_bufs[s], gsems[s])
            if k >= depth:
                p = (k - depth) % nbuf
                gathers[p].wait()
                stores[p] = pltpu.async_copy(
                    row_bufs[p],
                    out_hbm.at[pl.ds(base + (k - depth) * chunk, chunk)],
                    osems[p])
        for k in range(max(n_chunks - depth, 0), n_chunks):
            p = k % nbuf
            gathers[p].wait()
            stores[p] = pltpu.async_copy(
                row_bufs[p], out_hbm.at[pl.ds(base + k * chunk, chunk)],
                osems[p])
        for st in stores:
            if st is not None:
                st.wait()

    return sc_gather


def kernel(noise, order):
    n, ncols = order.shape
    b = noise.shape[0]
    flat_idx = _flat_indices(noise, n, ncols).reshape(-1)
    order_flat = order.reshape(-1)
    total = b * ncols
    out_flat = _make_sc_gather(total, 8192)(order_flat, flat_idx)
    return out_flat.reshape(b, ncols)


# speedup vs baseline: 1.0213x; 1.0213x over previous
import functools

import jax
import jax.numpy as jnp
from jax import lax
from jax.experimental import pallas as pl
from jax.experimental.pallas import tpu as pltpu
from jax.experimental.pallas import tpu_sc as plsc

_SC_INFO = plsc.get_sparse_core_info()
_NC = _SC_INFO.num_cores
_NS = _SC_INFO.num_subcores
_NW = _NC * _NS


def _idx_body(n, ncols, noise_ref, out_ref):
    x = noise_ref[...]
    unif = 0.5 * (1.0 + lax.erf(x / jnp.sqrt(jnp.asarray(2.0, x.dtype))))
    idx = jnp.floor(unif * n).astype(jnp.int32)
    idx = jnp.minimum(idx, n - 1)
    col = lax.broadcasted_iota(jnp.int32, x.shape, 1)
    out_ref[...] = idx * ncols + col


def _flat_indices(noise, n, ncols):
    b = noise.shape[0]
    block_rows = 8192
    grid = (b // block_rows,)
    return pl.pallas_call(
        functools.partial(_idx_body, n, ncols),
        grid=grid,
        in_specs=[pl.BlockSpec((block_rows, ncols), lambda i: (i, 0))],
        out_specs=pl.BlockSpec((block_rows, ncols), lambda i: (i, 0)),
        out_shape=jax.ShapeDtypeStruct((b, ncols), jnp.int32),
    )(noise)


def _make_sc_gather(total, chunk):
    per_w = total // _NW
    n_chunks = per_w // chunk
    mesh = plsc.VectorSubcoreMesh(core_axis_name="c", subcore_axis_name="s")

    nbuf = 4
    depth = 2

    @functools.partial(
        pl.kernel,
        mesh=mesh,
        out_type=jax.ShapeDtypeStruct((total,), jnp.float32),
        scratch_types=(
            [pltpu.VMEM((chunk,), jnp.int32) for _ in range(nbuf)]
            + [pltpu.VMEM((chunk,), jnp.float32) for _ in range(nbuf)]
            + [pltpu.SemaphoreType.DMA for _ in range(2 * nbuf)]
        ),
    )
    def sc_gather(order_hbm, idx_hbm, out_hbm, *bufs):
        idx_bufs = bufs[:nbuf]
        row_bufs = bufs[nbuf:2 * nbuf]
        gsems = bufs[2 * nbuf:3 * nbuf]
        osems = bufs[3 * nbuf:]
        wid = lax.axis_index("s") * _NC + lax.axis_index("c")
        base = wid * per_w
        gathers = [None] * nbuf
        stores = [None] * nbuf
        for k in range(n_chunks):
            s = k % nbuf
            if stores[s] is not None:
                stores[s].wait()
            pltpu.sync_copy(idx_hbm.at[pl.ds(base + k * chunk, chunk)],
                            idx_bufs[s])
            gathers[s] = pltpu.async_copy(
                order_hbm.at[idx_bufs[s]], row_bufs[s], gsems[s])
            if k >= depth:
                p = (k - depth) % nbuf
                gathers[p].wait()
                stores[p] = pltpu.async_copy(
                    row_bufs[p],
                    out_hbm.at[pl.ds(base + (k - depth) * chunk, chunk)],
                    osems[p])
        for k in range(max(n_chunks - depth, 0), n_chunks):
            p = k % nbuf
            gathers[p].wait()
            stores[p] = pltpu.async_copy(
                row_bufs[p], out_hbm.at[pl.ds(base + k * chunk, chunk)],
                osems[p])
        for st in stores:
            if st is not None:
                st.wait()

    return sc_gather


def kernel(noise, order):
    n, ncols = order.shape
    b = noise.shape[0]
    flat_idx = _flat_indices(noise, n, ncols).reshape(-1)
    order_flat = order.reshape(-1)
    total = b * ncols
    out_flat = _make_sc_gather(total, 8192)(order_flat, flat_idx)
    return out_flat.reshape(b, ncols)
